# baseline (speedup 1.0000x reference)
import jax
import jax.numpy as jnp
from jax.experimental import pallas as pl
from jax.experimental.pallas import tpu as pltpu

_TB = 256


def _sbq_kernel(z_ref, wi_ref, bi_ref, cbt_ref, wo_ref, bo_ref,
                zq_ref, codes_ref, lats_ref, loss_ref):
    nlayers = cbt_ref.shape[1]
    cd = cbt_ref.shape[2]
    cs = cbt_ref.shape[3]
    sub = z_ref.shape[1]
    t_total = z_ref.shape[2]

    iota_row = jax.lax.broadcasted_iota(jnp.int32, (1, cs), 1).astype(jnp.float32)
    cb_gathers, cbt_augs = [], []
    for l in range(nlayers):
        cbt = cbt_ref[0, l]
        norm = jnp.sqrt(jnp.sum(cbt * cbt, axis=0, keepdims=True))
        cbt_n = cbt / jnp.maximum(norm, 1e-12)
        c2 = jnp.sum(cbt_n * cbt_n, axis=0, keepdims=True)
        cb_gathers.append(jnp.concatenate([cbt, iota_row], axis=0))
        cbt_augs.append(jnp.concatenate([-2.0 * cbt_n, c2], axis=0))

    loss = jnp.float32(0.0)
    for ts in range(t_total // _TB):
        sl = pl.ds(ts * _TB, _TB)
        x = z_ref[0, :, sl]
        residual = x
        zq_acc = jnp.zeros_like(x)
        for l in range(nlayers):
            wi = wi_ref[0, l]
            bi = bi_ref[0, l]
            wo = wo_ref[0, l]
            bo = bo_ref[0, l]

            z_e = jnp.dot(wi, residual,
                          preferred_element_type=jnp.float32) + bi
            n = jnp.sqrt(jnp.sum(z_e * z_e, axis=0, keepdims=True))
            enc_n = z_e / jnp.maximum(n, 1e-12)
            enc_aug = jnp.concatenate(
                [enc_n, jnp.ones((1, enc_n.shape[1]), jnp.float32)],
                axis=0)

            q = jax.lax.dot_general(cbt_augs[l], enc_aug,
                                    (((0,), (0,)), ((), ())),
                                    preferred_element_type=jnp.float32)

            best = jnp.min(q, axis=0, keepdims=True)
            onehot = (q <= best).astype(jnp.float32)
            zq_aug = jnp.dot(cb_gathers[l], onehot,
                             preferred_element_type=jnp.float32)
            z_q = zq_aug[:cd]
            idx = zq_aug[cd:cd + 1].astype(jnp.int32)

            z_q_st = z_e + (z_q - z_e)
            out = jnp.dot(wo, z_q_st,
                          preferred_element_type=jnp.float32) + bo
            zq_acc = zq_acc + out
            residual = residual - out
            loss = loss + jnp.sum((z_e - z_q) ** 2)

            codes_ref[0, 0, l:l + 1, sl] = idx
            lats_ref[0, l * cd:(l + 1) * cd, sl] = z_e

        zq_ref[0, :, sl] = zq_acc
    loss_ref[0, 0, 0, 0] = loss


def kernel(z, W_in, b_in, codebook, W_out, b_out):
    B, C, T = z.shape
    G, L, CD, SUB = W_in.shape
    CS = codebook.shape[2]

    bi = b_in.reshape(G, L, CD, 1)
    bo = b_out.reshape(G, L, SUB, 1)
    cbt = codebook.transpose(0, 1, 3, 2)

    zq, codes_tmp, lats, loss_parts = pl.pallas_call(
        _sbq_kernel,
        grid=(G, B),
        in_specs=[
            pl.BlockSpec((1, SUB, T), lambda g, b: (b, g, 0)),
            pl.BlockSpec((1, L, CD, SUB), lambda g, b: (g, 0, 0, 0)),
            pl.BlockSpec((1, L, CD, 1), lambda g, b: (g, 0, 0, 0)),
            pl.BlockSpec((1, L, CD, CS), lambda g, b: (g, 0, 0, 0)),
            pl.BlockSpec((1, L, SUB, CD), lambda g, b: (g, 0, 0, 0)),
            pl.BlockSpec((1, L, SUB, 1), lambda g, b: (g, 0, 0, 0)),
        ],
        out_specs=[
            pl.BlockSpec((1, SUB, T), lambda g, b: (b, g, 0)),
            pl.BlockSpec((1, 1, L, T), lambda g, b: (g, b, 0, 0)),
            pl.BlockSpec((1, L * CD, T), lambda g, b: (b, g, 0)),
            pl.BlockSpec((1, 1, 1, 1), lambda g, b: (g, b, 0, 0),
                         memory_space=pltpu.SMEM),
        ],
        out_shape=[
            jax.ShapeDtypeStruct((B, C, T), jnp.float32),
            jax.ShapeDtypeStruct((G, B, L, T), jnp.int32),
            jax.ShapeDtypeStruct((B, G * L * CD, T), jnp.float32),
            jax.ShapeDtypeStruct((G, B, 1, 1), jnp.float32),
        ],
        compiler_params=pltpu.CompilerParams(
            dimension_semantics=("parallel", "parallel"),
        ),
    )(z, W_in, bi, cbt, W_out, bo)

    codes = codes_tmp.transpose(1, 0, 2, 3).reshape(B, G * L, T)
    total = jnp.sum(loss_parts) / jnp.float32(G * B * CD * T)
    return zq, codes, lats, total, total


# speedup vs baseline: 2.5926x; 2.5926x over previous
import jax
import jax.numpy as jnp
from jax.experimental import pallas as pl
from jax.experimental.pallas import tpu as pltpu

_TB = 1024


def _sbq_kernel(z_ref, wi_ref, bi_ref, cbt_ref, wo_ref, bo_ref,
                zq_ref, codes_ref, lats_ref, loss_ref):
    nlayers = cbt_ref.shape[1]
    cd = cbt_ref.shape[2]
    cs = cbt_ref.shape[3]
    sub = z_ref.shape[1]
    t_total = z_ref.shape[2]

    iota_row = jax.lax.broadcasted_iota(jnp.int32, (1, cs), 1).astype(jnp.float32)
    cb_gathers, cbt_augs = [], []
    for l in range(nlayers):
        cbt = cbt_ref[0, l]
        norm = jnp.sqrt(jnp.sum(cbt * cbt, axis=0, keepdims=True))
        cbt_n = cbt / jnp.maximum(norm, 1e-12)
        c2 = jnp.sum(cbt_n * cbt_n, axis=0, keepdims=True)
        cb_gathers.append(jnp.concatenate([cbt, iota_row], axis=0))
        cbt_augs.append(jnp.concatenate([-2.0 * cbt_n, c2], axis=0))

    loss = jnp.float32(0.0)
    for ts in range(t_total // _TB):
        sl = pl.ds(ts * _TB, _TB)
        x = z_ref[0, :, sl]
        residual = x
        zq_acc = jnp.zeros_like(x)
        for l in range(nlayers):
            wi = wi_ref[0, l]
            bi = bi_ref[0, l]
            wo = wo_ref[0, l]
            bo = bo_ref[0, l]

            z_e = jnp.dot(wi, residual,
                          preferred_element_type=jnp.float32) + bi
            n = jnp.sqrt(jnp.sum(z_e * z_e, axis=0, keepdims=True))
            enc_n = z_e / jnp.maximum(n, 1e-12)
            enc_aug = jnp.concatenate(
                [enc_n, jnp.ones((1, enc_n.shape[1]), jnp.float32)],
                axis=0)

            q = jax.lax.dot_general(cbt_augs[l], enc_aug,
                                    (((0,), (0,)), ((), ())),
                                    preferred_element_type=jnp.float32)

            best = jnp.min(q, axis=0, keepdims=True)
            onehot = (q <= best).astype(jnp.float32)
            zq_aug = jnp.dot(cb_gathers[l], onehot,
                             preferred_element_type=jnp.float32)
            z_q = zq_aug[:cd]
            idx = zq_aug[cd:cd + 1].astype(jnp.int32)

            z_q_st = z_e + (z_q - z_e)
            out = jnp.dot(wo, z_q_st,
                          preferred_element_type=jnp.float32) + bo
            zq_acc = zq_acc + out
            residual = residual - out
            loss = loss + jnp.sum((z_e - z_q) ** 2)

            codes_ref[0, 0, l:l + 1, sl] = idx
            lats_ref[0, l * cd:(l + 1) * cd, sl] = z_e

        zq_ref[0, :, sl] = zq_acc
    loss_ref[0, 0, 0, 0] = loss


def kernel(z, W_in, b_in, codebook, W_out, b_out):
    B, C, T = z.shape
    G, L, CD, SUB = W_in.shape
    CS = codebook.shape[2]

    bi = b_in.reshape(G, L, CD, 1)
    bo = b_out.reshape(G, L, SUB, 1)
    cbt = codebook.transpose(0, 1, 3, 2)

    zq, codes_tmp, lats, loss_parts = pl.pallas_call(
        _sbq_kernel,
        grid=(G, B),
        in_specs=[
            pl.BlockSpec((1, SUB, T), lambda g, b: (b, g, 0)),
            pl.BlockSpec((1, L, CD, SUB), lambda g, b: (g, 0, 0, 0)),
            pl.BlockSpec((1, L, CD, 1), lambda g, b: (g, 0, 0, 0)),
            pl.BlockSpec((1, L, CD, CS), lambda g, b: (g, 0, 0, 0)),
            pl.BlockSpec((1, L, SUB, CD), lambda g, b: (g, 0, 0, 0)),
            pl.BlockSpec((1, L, SUB, 1), lambda g, b: (g, 0, 0, 0)),
        ],
        out_specs=[
            pl.BlockSpec((1, SUB, T), lambda g, b: (b, g, 0)),
            pl.BlockSpec((1, 1, L, T), lambda g, b: (g, b, 0, 0)),
            pl.BlockSpec((1, L * CD, T), lambda g, b: (b, g, 0)),
            pl.BlockSpec((1, 1, 1, 1), lambda g, b: (g, b, 0, 0),
                         memory_space=pltpu.SMEM),
        ],
        out_shape=[
            jax.ShapeDtypeStruct((B, C, T), jnp.float32),
            jax.ShapeDtypeStruct((G, B, L, T), jnp.int32),
            jax.ShapeDtypeStruct((B, G * L * CD, T), jnp.float32),
            jax.ShapeDtypeStruct((G, B, 1, 1), jnp.float32),
        ],
        compiler_params=pltpu.CompilerParams(
            dimension_semantics=("parallel", "parallel"),
        ),
    )(z, W_in, bi, cbt, W_out, bo)

    codes = codes_tmp.transpose(1, 0, 2, 3).reshape(B, G * L, T)
    total = jnp.sum(loss_parts) / jnp.float32(G * B * CD * T)
    return zq, codes, lats, total, total


# speedup vs baseline: 2.9424x; 1.1349x over previous
import jax
import jax.numpy as jnp
from jax.experimental import pallas as pl
from jax.experimental.pallas import tpu as pltpu

_TB = 2048


def _sbq_kernel(z_ref, wi_ref, bi_ref, cbt_ref, wo_ref, bo_ref,
                zq_ref, codes_ref, lats_ref, loss_ref):
    nlayers = cbt_ref.shape[1]
    cd = cbt_ref.shape[2]
    cs = cbt_ref.shape[3]
    sub = z_ref.shape[1]
    t_total = z_ref.shape[2]

    iota_row = jax.lax.broadcasted_iota(jnp.int32, (1, cs), 1).astype(jnp.float32)
    cb_gathers, cbt_augs = [], []
    for l in range(nlayers):
        cbt = cbt_ref[0, l]
        norm = jnp.sqrt(jnp.sum(cbt * cbt, axis=0, keepdims=True))
        cbt_n = cbt / jnp.maximum(norm, 1e-12)
        c2 = jnp.sum(cbt_n * cbt_n, axis=0, keepdims=True)
        cb_gathers.append(jnp.concatenate([cbt, iota_row], axis=0))
        cbt_augs.append(jnp.concatenate([-2.0 * cbt_n, c2], axis=0))

    loss = jnp.float32(0.0)
    for ts in range(t_total // _TB):
        sl = pl.ds(ts * _TB, _TB)
        x = z_ref[0, :, sl]
        residual = x
        zq_acc = jnp.zeros_like(x)
        for l in range(nlayers):
            wi = wi_ref[0, l]
            bi = bi_ref[0, l]
            wo = wo_ref[0, l]
            bo = bo_ref[0, l]

            z_e = jnp.dot(wi, residual,
                          preferred_element_type=jnp.float32) + bi
            n = jnp.sqrt(jnp.sum(z_e * z_e, axis=0, keepdims=True))
            enc_n = z_e / jnp.maximum(n, 1e-12)
            enc_aug = jnp.concatenate(
                [enc_n, jnp.ones((1, enc_n.shape[1]), jnp.float32)],
                axis=0)

            q = jax.lax.dot_general(cbt_augs[l], enc_aug,
                                    (((0,), (0,)), ((), ())),
                                    preferred_element_type=jnp.float32)

            best = jnp.min(q, axis=0, keepdims=True)
            onehot = (q <= best).astype(jnp.float32)
            zq_aug = jnp.dot(cb_gathers[l], onehot,
                             preferred_element_type=jnp.float32)
            z_q = zq_aug[:cd]
            idx = zq_aug[cd:cd + 1].astype(jnp.int32)

            z_q_st = z_e + (z_q - z_e)
            out = jnp.dot(wo, z_q_st,
                          preferred_element_type=jnp.float32) + bo
            zq_acc = zq_acc + out
            residual = residual - out
            loss = loss + jnp.sum((z_e - z_q) ** 2)

            codes_ref[0, 0, l:l + 1, sl] = idx
            lats_ref[0, l * cd:(l + 1) * cd, sl] = z_e

        zq_ref[0, :, sl] = zq_acc
    loss_ref[0, 0, 0, 0] = loss


def kernel(z, W_in, b_in, codebook, W_out, b_out):
    B, C, T = z.shape
    G, L, CD, SUB = W_in.shape
    CS = codebook.shape[2]

    bi = b_in.reshape(G, L, CD, 1)
    bo = b_out.reshape(G, L, SUB, 1)
    cbt = codebook.transpose(0, 1, 3, 2)

    zq, codes_tmp, lats, loss_parts = pl.pallas_call(
        _sbq_kernel,
        grid=(G, B),
        in_specs=[
            pl.BlockSpec((1, SUB, T), lambda g, b: (b, g, 0)),
            pl.BlockSpec((1, L, CD, SUB), lambda g, b: (g, 0, 0, 0)),
            pl.BlockSpec((1, L, CD, 1), lambda g, b: (g, 0, 0, 0)),
            pl.BlockSpec((1, L, CD, CS), lambda g, b: (g, 0, 0, 0)),
            pl.BlockSpec((1, L, SUB, CD), lambda g, b: (g, 0, 0, 0)),
            pl.BlockSpec((1, L, SUB, 1), lambda g, b: (g, 0, 0, 0)),
        ],
        out_specs=[
            pl.BlockSpec((1, SUB, T), lambda g, b: (b, g, 0)),
            pl.BlockSpec((1, 1, L, T), lambda g, b: (g, b, 0, 0)),
            pl.BlockSpec((1, L * CD, T), lambda g, b: (b, g, 0)),
            pl.BlockSpec((1, 1, 1, 1), lambda g, b: (g, b, 0, 0),
                         memory_space=pltpu.SMEM),
        ],
        out_shape=[
            jax.ShapeDtypeStruct((B, C, T), jnp.float32),
            jax.ShapeDtypeStruct((G, B, L, T), jnp.int32),
            jax.ShapeDtypeStruct((B, G * L * CD, T), jnp.float32),
            jax.ShapeDtypeStruct((G, B, 1, 1), jnp.float32),
        ],
        compiler_params=pltpu.CompilerParams(
            dimension_semantics=("parallel", "parallel"),
        ),
    )(z, W_in, bi, cbt, W_out, bo)

    codes = codes_tmp.transpose(1, 0, 2, 3).reshape(B, G * L, T)
    total = jnp.sum(loss_parts) / jnp.float32(G * B * CD * T)
    return zq, codes, lats, total, total


# speedup vs baseline: 3.0537x; 1.0378x over previous
import jax
import jax.numpy as jnp
from jax.experimental import pallas as pl
from jax.experimental.pallas import tpu as pltpu

_TB = 2048
_BB = 2


def _sbq_kernel(z_ref, wi_ref, bi_ref, cbt_ref, wo_ref, bo_ref,
                zq_ref, codes_ref, lats_ref, loss_ref):
    nlayers = cbt_ref.shape[1]
    cd = cbt_ref.shape[2]
    cs = cbt_ref.shape[3]
    sub = z_ref.shape[1]
    t_total = z_ref.shape[2]

    iota_row = jax.lax.broadcasted_iota(jnp.int32, (1, cs), 1).astype(jnp.float32)
    cb_gathers, cbt_augs = [], []
    for l in range(nlayers):
        cbt = cbt_ref[0, l]
        norm = jnp.sqrt(jnp.sum(cbt * cbt, axis=0, keepdims=True))
        cbt_n = cbt / jnp.maximum(norm, 1e-12)
        c2 = jnp.sum(cbt_n * cbt_n, axis=0, keepdims=True)
        cb_gathers.append(jnp.concatenate([cbt, iota_row], axis=0))
        cbt_augs.append(jnp.concatenate([-2.0 * cbt_n, c2], axis=0))

    nb = z_ref.shape[0]
    for bb in range(nb):
      loss = jnp.float32(0.0)
      for ts in range(t_total // _TB):
        sl = pl.ds(ts * _TB, _TB)
        x = z_ref[bb, :, sl]
        residual = x
        zq_acc = jnp.zeros_like(x)
        for l in range(nlayers):
            wi = wi_ref[0, l]
            bi = bi_ref[0, l]
            wo = wo_ref[0, l]
            bo = bo_ref[0, l]

            z_e = jnp.dot(wi, residual,
                          preferred_element_type=jnp.float32) + bi
            n = jnp.sqrt(jnp.sum(z_e * z_e, axis=0, keepdims=True))
            enc_n = z_e / jnp.maximum(n, 1e-12)
            enc_aug = jnp.concatenate(
                [enc_n, jnp.ones((1, enc_n.shape[1]), jnp.float32)],
                axis=0)

            q = jax.lax.dot_general(cbt_augs[l], enc_aug,
                                    (((0,), (0,)), ((), ())),
                                    preferred_element_type=jnp.float32)

            best = jnp.min(q, axis=0, keepdims=True)
            onehot = (q <= best).astype(jnp.float32)
            zq_aug = jnp.dot(cb_gathers[l], onehot,
                             preferred_element_type=jnp.float32)
            z_q = zq_aug[:cd]
            idx = zq_aug[cd:cd + 1].astype(jnp.int32)

            z_q_st = z_e + (z_q - z_e)
            out = jnp.dot(wo, z_q_st,
                          preferred_element_type=jnp.float32) + bo
            zq_acc = zq_acc + out
            residual = residual - out
            loss = loss + jnp.sum((z_e - z_q) ** 2)

            codes_ref[0, bb, l:l + 1, sl] = idx
            lats_ref[bb, l * cd:(l + 1) * cd, sl] = z_e

        zq_ref[bb, :, sl] = zq_acc
      loss_ref[0, bb, 0, 0] = loss


def kernel(z, W_in, b_in, codebook, W_out, b_out):
    B, C, T = z.shape
    G, L, CD, SUB = W_in.shape
    CS = codebook.shape[2]

    bi = b_in.reshape(G, L, CD, 1)
    bo = b_out.reshape(G, L, SUB, 1)
    cbt = codebook.transpose(0, 1, 3, 2)

    zq, codes_tmp, lats, loss_parts = pl.pallas_call(
        _sbq_kernel,
        grid=(G, B // _BB),
        in_specs=[
            pl.BlockSpec((_BB, SUB, T), lambda g, b: (b, g, 0)),
            pl.BlockSpec((1, L, CD, SUB), lambda g, b: (g, 0, 0, 0)),
            pl.BlockSpec((1, L, CD, 1), lambda g, b: (g, 0, 0, 0)),
            pl.BlockSpec((1, L, CD, CS), lambda g, b: (g, 0, 0, 0)),
            pl.BlockSpec((1, L, SUB, CD), lambda g, b: (g, 0, 0, 0)),
            pl.BlockSpec((1, L, SUB, 1), lambda g, b: (g, 0, 0, 0)),
        ],
        out_specs=[
            pl.BlockSpec((_BB, SUB, T), lambda g, b: (b, g, 0)),
            pl.BlockSpec((1, _BB, L, T), lambda g, b: (g, b, 0, 0)),
            pl.BlockSpec((_BB, L * CD, T), lambda g, b: (b, g, 0)),
            pl.BlockSpec((1, _BB, 1, 1), lambda g, b: (g, b, 0, 0),
                         memory_space=pltpu.SMEM),
        ],
        out_shape=[
            jax.ShapeDtypeStruct((B, C, T), jnp.float32),
            jax.ShapeDtypeStruct((G, B, L, T), jnp.int32),
            jax.ShapeDtypeStruct((B, G * L * CD, T), jnp.float32),
            jax.ShapeDtypeStruct((G, B, 1, 1), jnp.float32),
        ],
        compiler_params=pltpu.CompilerParams(
            dimension_semantics=("parallel", "parallel"),
        ),
    )(z, W_in, bi, cbt, W_out, bo)

    codes = codes_tmp.transpose(1, 0, 2, 3).reshape(B, G * L, T)
    total = jnp.sum(loss_parts) / jnp.float32(G * B * CD * T)
    return zq, codes, lats, total, total


# speedup vs baseline: 3.1184x; 1.0212x over previous
import jax
import jax.numpy as jnp
from jax.experimental import pallas as pl
from jax.experimental.pallas import tpu as pltpu

_TB = 2048
_BB = 4


def _sbq_kernel(z_ref, wi_ref, bi_ref, cbt_ref, wo_ref, bo_ref,
                zq_ref, codes_ref, lats_ref, loss_ref):
    nlayers = cbt_ref.shape[1]
    cd = cbt_ref.shape[2]
    cs = cbt_ref.shape[3]
    sub = z_ref.shape[1]
    t_total = z_ref.shape[2]

    iota_row = jax.lax.broadcasted_iota(jnp.int32, (1, cs), 1).astype(jnp.float32)
    cb_gathers, cbt_augs = [], []
    for l in range(nlayers):
        cbt = cbt_ref[0, l]
        norm = jnp.sqrt(jnp.sum(cbt * cbt, axis=0, keepdims=True))
        cbt_n = cbt / jnp.maximum(norm, 1e-12)
        c2 = jnp.sum(cbt_n * cbt_n, axis=0, keepdims=True)
        cb_gathers.append(jnp.concatenate([cbt, iota_row], axis=0))
        cbt_augs.append(jnp.concatenate([-2.0 * cbt_n, c2], axis=0))

    nb = z_ref.shape[0]
    for bb in range(nb):
      loss = jnp.float32(0.0)
      for ts in range(t_total // _TB):
        sl = pl.ds(ts * _TB, _TB)
        x = z_ref[bb, :, sl]
        residual = x
        zq_acc = jnp.zeros_like(x)
        for l in range(nlayers):
            wi = wi_ref[0, l]
            bi = bi_ref[0, l]
            wo = wo_ref[0, l]
            bo = bo_ref[0, l]

            z_e = jnp.dot(wi, residual,
                          preferred_element_type=jnp.float32) + bi
            n = jnp.sqrt(jnp.sum(z_e * z_e, axis=0, keepdims=True))
            enc_n = z_e / jnp.maximum(n, 1e-12)
            enc_aug = jnp.concatenate(
                [enc_n, jnp.ones((1, enc_n.shape[1]), jnp.float32)],
                axis=0)

            q = jax.lax.dot_general(cbt_augs[l], enc_aug,
                                    (((0,), (0,)), ((), ())),
                                    preferred_element_type=jnp.float32)

            best = jnp.min(q, axis=0, keepdims=True)
            onehot = (q <= best).astype(jnp.float32)
            zq_aug = jnp.dot(cb_gathers[l], onehot,
                             preferred_element_type=jnp.float32)
            z_q = zq_aug[:cd]
            idx = zq_aug[cd:cd + 1].astype(jnp.int32)

            z_q_st = z_e + (z_q - z_e)
            out = jnp.dot(wo, z_q_st,
                          preferred_element_type=jnp.float32) + bo
            zq_acc = zq_acc + out
            residual = residual - out
            loss = loss + jnp.sum((z_e - z_q) ** 2)

            codes_ref[0, bb, l:l + 1, sl] = idx
            lats_ref[bb, l * cd:(l + 1) * cd, sl] = z_e

        zq_ref[bb, :, sl] = zq_acc
      loss_ref[0, bb, 0, 0] = loss


def kernel(z, W_in, b_in, codebook, W_out, b_out):
    B, C, T = z.shape
    G, L, CD, SUB = W_in.shape
    CS = codebook.shape[2]

    bi = b_in.reshape(G, L, CD, 1)
    bo = b_out.reshape(G, L, SUB, 1)
    cbt = codebook.transpose(0, 1, 3, 2)

    zq, codes_tmp, lats, loss_parts = pl.pallas_call(
        _sbq_kernel,
        grid=(G, B // _BB),
        in_specs=[
            pl.BlockSpec((_BB, SUB, T), lambda g, b: (b, g, 0)),
            pl.BlockSpec((1, L, CD, SUB), lambda g, b: (g, 0, 0, 0)),
            pl.BlockSpec((1, L, CD, 1), lambda g, b: (g, 0, 0, 0)),
            pl.BlockSpec((1, L, CD, CS), lambda g, b: (g, 0, 0, 0)),
            pl.BlockSpec((1, L, SUB, CD), lambda g, b: (g, 0, 0, 0)),
            pl.BlockSpec((1, L, SUB, 1), lambda g, b: (g, 0, 0, 0)),
        ],
        out_specs=[
            pl.BlockSpec((_BB, SUB, T), lambda g, b: (b, g, 0)),
            pl.BlockSpec((1, _BB, L, T), lambda g, b: (g, b, 0, 0)),
            pl.BlockSpec((_BB, L * CD, T), lambda g, b: (b, g, 0)),
            pl.BlockSpec((1, _BB, 1, 1), lambda g, b: (g, b, 0, 0),
                         memory_space=pltpu.SMEM),
        ],
        out_shape=[
            jax.ShapeDtypeStruct((B, C, T), jnp.float32),
            jax.ShapeDtypeStruct((G, B, L, T), jnp.int32),
            jax.ShapeDtypeStruct((B, G * L * CD, T), jnp.float32),
            jax.ShapeDtypeStruct((G, B, 1, 1), jnp.float32),
        ],
        compiler_params=pltpu.CompilerParams(
            dimension_semantics=("parallel", "parallel"),
        ),
    )(z, W_in, bi, cbt, W_out, bo)

    codes = codes_tmp.transpose(1, 0, 2, 3).reshape(B, G * L, T)
    total = jnp.sum(loss_parts) / jnp.float32(G * B * CD * T)
    return zq, codes, lats, total, total


# speedup vs baseline: 3.1425x; 1.0077x over previous
import jax
import jax.numpy as jnp
from jax.experimental import pallas as pl
from jax.experimental.pallas import tpu as pltpu

_TB = 2048
_BB = 8


def _sbq_kernel(z_ref, wi_ref, bi_ref, cbt_ref, wo_ref, bo_ref,
                zq_ref, codes_ref, lats_ref, loss_ref):
    nlayers = cbt_ref.shape[1]
    cd = cbt_ref.shape[2]
    cs = cbt_ref.shape[3]
    sub = z_ref.shape[1]
    t_total = z_ref.shape[2]

    iota_row = jax.lax.broadcasted_iota(jnp.int32, (1, cs), 1).astype(jnp.float32)
    cb_gathers, cbt_augs = [], []
    for l in range(nlayers):
        cbt = cbt_ref[0, l]
        norm = jnp.sqrt(jnp.sum(cbt * cbt, axis=0, keepdims=True))
        cbt_n = cbt / jnp.maximum(norm, 1e-12)
        c2 = jnp.sum(cbt_n * cbt_n, axis=0, keepdims=True)
        cb_gathers.append(jnp.concatenate([cbt, iota_row], axis=0))
        cbt_augs.append(jnp.concatenate([-2.0 * cbt_n, c2], axis=0))

    nb = z_ref.shape[0]
    for bb in range(nb):
      loss = jnp.float32(0.0)
      for ts in range(t_total // _TB):
        sl = pl.ds(ts * _TB, _TB)
        x = z_ref[bb, :, sl]
        residual = x
        zq_acc = jnp.zeros_like(x)
        for l in range(nlayers):
            wi = wi_ref[0, l]
            bi = bi_ref[0, l]
            wo = wo_ref[0, l]
            bo = bo_ref[0, l]

            z_e = jnp.dot(wi, residual,
                          preferred_element_type=jnp.float32) + bi
            n = jnp.sqrt(jnp.sum(z_e * z_e, axis=0, keepdims=True))
            enc_n = z_e / jnp.maximum(n, 1e-12)
            enc_aug = jnp.concatenate(
                [enc_n, jnp.ones((1, enc_n.shape[1]), jnp.float32)],
                axis=0)

            q = jax.lax.dot_general(cbt_augs[l], enc_aug,
                                    (((0,), (0,)), ((), ())),
                                    preferred_element_type=jnp.float32)

            best = jnp.min(q, axis=0, keepdims=True)
            onehot = (q <= best).astype(jnp.float32)
            zq_aug = jnp.dot(cb_gathers[l], onehot,
                             preferred_element_type=jnp.float32)
            z_q = zq_aug[:cd]
            idx = zq_aug[cd:cd + 1].astype(jnp.int32)

            z_q_st = z_e + (z_q - z_e)
            out = jnp.dot(wo, z_q_st,
                          preferred_element_type=jnp.float32) + bo
            zq_acc = zq_acc + out
            residual = residual - out
            loss = loss + jnp.sum((z_e - z_q) ** 2)

            codes_ref[0, bb, l:l + 1, sl] = idx
            lats_ref[bb, l * cd:(l + 1) * cd, sl] = z_e

        zq_ref[bb, :, sl] = zq_acc
      loss_ref[0, bb, 0, 0] = loss


def kernel(z, W_in, b_in, codebook, W_out, b_out):
    B, C, T = z.shape
    G, L, CD, SUB = W_in.shape
    CS = codebook.shape[2]

    bi = b_in.reshape(G, L, CD, 1)
    bo = b_out.reshape(G, L, SUB, 1)
    cbt = codebook.transpose(0, 1, 3, 2)

    zq, codes_tmp, lats, loss_parts = pl.pallas_call(
        _sbq_kernel,
        grid=(G, B // _BB),
        in_specs=[
            pl.BlockSpec((_BB, SUB, T), lambda g, b: (b, g, 0)),
            pl.BlockSpec((1, L, CD, SUB), lambda g, b: (g, 0, 0, 0)),
            pl.BlockSpec((1, L, CD, 1), lambda g, b: (g, 0, 0, 0)),
            pl.BlockSpec((1, L, CD, CS), lambda g, b: (g, 0, 0, 0)),
            pl.BlockSpec((1, L, SUB, CD), lambda g, b: (g, 0, 0, 0)),
            pl.BlockSpec((1, L, SUB, 1), lambda g, b: (g, 0, 0, 0)),
        ],
        out_specs=[
            pl.BlockSpec((_BB, SUB, T), lambda g, b: (b, g, 0)),
            pl.BlockSpec((1, _BB, L, T), lambda g, b: (g, b, 0, 0)),
            pl.BlockSpec((_BB, L * CD, T), lambda g, b: (b, g, 0)),
            pl.BlockSpec((1, _BB, 1, 1), lambda g, b: (g, b, 0, 0),
                         memory_space=pltpu.SMEM),
        ],
        out_shape=[
            jax.ShapeDtypeStruct((B, C, T), jnp.float32),
            jax.ShapeDtypeStruct((G, B, L, T), jnp.int32),
            jax.ShapeDtypeStruct((B, G * L * CD, T), jnp.float32),
            jax.ShapeDtypeStruct((G, B, 1, 1), jnp.float32),
        ],
        compiler_params=pltpu.CompilerParams(
            dimension_semantics=("parallel", "parallel"),
        ),
    )(z, W_in, bi, cbt, W_out, bo)

    codes = codes_tmp.transpose(1, 0, 2, 3).reshape(B, G * L, T)
    total = jnp.sum(loss_parts) / jnp.float32(G * B * CD * T)
    return zq, codes, lats, total, total


# speedup vs baseline: 3.1779x; 1.0113x over previous
import jax
import jax.numpy as jnp
from jax.experimental import pallas as pl
from jax.experimental.pallas import tpu as pltpu

_TB = 2048
_BB = 8


def _sbq_kernel(z_ref, wi_ref, bi_ref, cbt_ref, wo_ref, bo_ref,
                zq_ref, codes_ref, lats_ref, loss_ref):
    nlayers = cbt_ref.shape[1]
    cd = cbt_ref.shape[2]
    cs = cbt_ref.shape[3]
    sub = z_ref.shape[1]
    t_total = z_ref.shape[2]

    iota_row = jax.lax.broadcasted_iota(jnp.int32, (1, cs), 1).astype(jnp.float32)
    cb_gathers, cbt_augs = [], []
    for l in range(nlayers):
        cbt = cbt_ref[0, l]
        norm = jnp.sqrt(jnp.sum(cbt * cbt, axis=0, keepdims=True))
        cbt_n = cbt / jnp.maximum(norm, 1e-12)
        cb_gathers.append(jnp.concatenate([cbt, iota_row], axis=0))
        cbt_augs.append(-cbt_n)

    nb = z_ref.shape[0]
    for bb in range(nb):
      loss = jnp.float32(0.0)
      for ts in range(t_total // _TB):
        sl = pl.ds(ts * _TB, _TB)
        x = z_ref[bb, :, sl]
        residual = x
        zq_acc = jnp.zeros_like(x)
        for l in range(nlayers):
            wi = wi_ref[0, l]
            bi = bi_ref[0, l]
            wo = wo_ref[0, l]
            bo = bo_ref[0, l]

            z_e = jnp.dot(wi, residual,
                          preferred_element_type=jnp.float32) + bi

            q = jax.lax.dot_general(cbt_augs[l], z_e,
                                    (((0,), (0,)), ((), ())),
                                    preferred_element_type=jnp.float32)

            best = jnp.min(q, axis=0, keepdims=True)
            onehot = (q <= best).astype(jnp.float32)
            zq_aug = jnp.dot(cb_gathers[l], onehot,
                             preferred_element_type=jnp.float32)
            z_q = zq_aug[:cd]
            idx = zq_aug[cd:cd + 1].astype(jnp.int32)

            z_q_st = z_e + (z_q - z_e)
            out = jnp.dot(wo, z_q_st,
                          preferred_element_type=jnp.float32) + bo
            zq_acc = zq_acc + out
            residual = residual - out
            loss = loss + jnp.sum((z_e - z_q) ** 2)

            codes_ref[0, bb, l:l + 1, sl] = idx
            lats_ref[bb, l * cd:(l + 1) * cd, sl] = z_e

        zq_ref[bb, :, sl] = zq_acc
      loss_ref[0, bb, 0, 0] = loss


def kernel(z, W_in, b_in, codebook, W_out, b_out):
    B, C, T = z.shape
    G, L, CD, SUB = W_in.shape
    CS = codebook.shape[2]

    bi = b_in.reshape(G, L, CD, 1)
    bo = b_out.reshape(G, L, SUB, 1)
    cbt = codebook.transpose(0, 1, 3, 2)

    zq, codes_tmp, lats, loss_parts = pl.pallas_call(
        _sbq_kernel,
        grid=(G, B // _BB),
        in_specs=[
            pl.BlockSpec((_BB, SUB, T), lambda g, b: (b, g, 0)),
            pl.BlockSpec((1, L, CD, SUB), lambda g, b: (g, 0, 0, 0)),
            pl.BlockSpec((1, L, CD, 1), lambda g, b: (g, 0, 0, 0)),
            pl.BlockSpec((1, L, CD, CS), lambda g, b: (g, 0, 0, 0)),
            pl.BlockSpec((1, L, SUB, CD), lambda g, b: (g, 0, 0, 0)),
            pl.BlockSpec((1, L, SUB, 1), lambda g, b: (g, 0, 0, 0)),
        ],
        out_specs=[
            pl.BlockSpec((_BB, SUB, T), lambda g, b: (b, g, 0)),
            pl.BlockSpec((1, _BB, L, T), lambda g, b: (g, b, 0, 0)),
            pl.BlockSpec((_BB, L * CD, T), lambda g, b: (b, g, 0)),
            pl.BlockSpec((1, _BB, 1, 1), lambda g, b: (g, b, 0, 0),
                         memory_space=pltpu.SMEM),
        ],
        out_shape=[
            jax.ShapeDtypeStruct((B, C, T), jnp.float32),
            jax.ShapeDtypeStruct((G, B, L, T), jnp.int32),
            jax.ShapeDtypeStruct((B, G * L * CD, T), jnp.float32),
            jax.ShapeDtypeStruct((G, B, 1, 1), jnp.float32),
        ],
        compiler_params=pltpu.CompilerParams(
            dimension_semantics=("parallel", "parallel"),
        ),
    )(z, W_in, bi, cbt, W_out, bo)

    codes = codes_tmp.transpose(1, 0, 2, 3).reshape(B, G * L, T)
    total = jnp.sum(loss_parts) / jnp.float32(G * B * CD * T)
    return zq, codes, lats, total, total
